# Initial kernel scaffold; baseline (speedup 1.0000x reference)
import jax
import jax.numpy as jnp
from jax.experimental import pallas as pl


def kernel(obs):
    raise NotImplementedError("write your pallas kernel here")


# speedup vs baseline: 12.4013x; 12.4013x over previous
import jax
import jax.numpy as jnp
from jax.experimental import pallas as pl
from jax.experimental.pallas import tpu as pltpu

_K = 10
_BLK = 512


def _pbe_body(x_ref, y_ref, out_ref):
    x = x_ref[...]
    y = y_ref[...]
    n = y.shape[0]
    x2 = jnp.sum(x * x, axis=1, keepdims=True)
    y2 = jnp.sum(y * y, axis=1)[None, :]
    xyt = jax.lax.dot_general(
        x, y, (((1,), (1,)), ((), ())),
        preferred_element_type=jnp.float32)
    d2 = jnp.maximum(x2 + y2 - 2.0 * xyt, 0.0)

    col = jax.lax.broadcasted_iota(jnp.int32, d2.shape, 1)
    acc = jnp.zeros((d2.shape[0],), jnp.float32)
    vals = d2
    for i in range(_K + 1):
        m = jnp.min(vals, axis=1)
        if i > 0:
            acc = acc + jnp.sqrt(m)
        if i < _K:
            is_min = vals <= m[:, None]
            idx = jnp.min(jnp.where(is_min, col, n), axis=1)
            vals = jnp.where(col == idx[:, None], jnp.inf, vals)
    out_ref[...] = jnp.log1p(acc * (1.0 / _K))[None, None, :]


def kernel(obs):
    x = obs
    if x.ndim == 1:
        x = x[:, None]
    b, d = x.shape
    nblk = b // _BLK
    out = pl.pallas_call(
        _pbe_body,
        grid=(nblk,),
        in_specs=[
            pl.BlockSpec((_BLK, d), lambda i: (i, 0)),
            pl.BlockSpec((b, d), lambda i: (0, 0)),
        ],
        out_specs=pl.BlockSpec((1, 1, _BLK), lambda i: (i, 0, 0)),
        out_shape=jax.ShapeDtypeStruct((nblk, 1, _BLK), jnp.float32),
    )(x, x)
    return out.reshape(b)


# speedup vs baseline: 21.8233x; 1.7598x over previous
import jax
import jax.numpy as jnp
from jax.experimental import pallas as pl
from jax.experimental.pallas import tpu as pltpu

_K = 10
_BLK = 512
_C = 128


def _pbe_body(x_ref, y_ref, out_ref):
    x = x_ref[...]
    y = y_ref[...]
    n = y.shape[0]
    r = x.shape[0]
    x2 = jnp.sum(x * x, axis=1, keepdims=True)
    y2 = jnp.sum(y * y, axis=1)[None, :]
    xyt = jax.lax.dot_general(
        x, y, (((1,), (1,)), ((), ())),
        preferred_element_type=jnp.float32)
    d2 = jnp.maximum(x2 + y2 - 2.0 * xyt, 0.0)

    nk = _K + 1
    ms = [jnp.full((r, _C), jnp.inf, jnp.float32) for _ in range(nk)]
    for c in range(n // _C):
        v = d2[:, c * _C:(c + 1) * _C]
        for i in range(nk):
            lo = jnp.minimum(ms[i], v)
            v = jnp.maximum(ms[i], v)
            ms[i] = lo

    lane = jax.lax.broadcasted_iota(jnp.int32, (r, _C), 1)
    acc = jnp.zeros((r,), jnp.float32)
    for t in range(nk):
        m = jnp.min(ms[0], axis=1)
        if t > 0:
            acc = acc + jnp.sqrt(m)
        if t < nk - 1:
            is_min = ms[0] <= m[:, None]
            idx = jnp.min(jnp.where(is_min, lane, _C), axis=1)
            pop = lane == idx[:, None]
            for i in range(nk - 1):
                ms[i] = jnp.where(pop, ms[i + 1], ms[i])
            ms[nk - 1] = jnp.where(pop, jnp.inf, ms[nk - 1])
    out_ref[...] = jnp.log1p(acc * (1.0 / _K))[None, None, :]


def kernel(obs):
    x = obs
    if x.ndim == 1:
        x = x[:, None]
    b, d = x.shape
    nblk = b // _BLK
    out = pl.pallas_call(
        _pbe_body,
        grid=(nblk,),
        in_specs=[
            pl.BlockSpec((_BLK, d), lambda i: (i, 0)),
            pl.BlockSpec((b, d), lambda i: (0, 0)),
        ],
        out_specs=pl.BlockSpec((1, 1, _BLK), lambda i: (i, 0, 0)),
        out_shape=jax.ShapeDtypeStruct((nblk, 1, _BLK), jnp.float32),
    )(x, x)
    return out.reshape(b)
